# Initial kernel scaffold; baseline (speedup 1.0000x reference)
import jax
import jax.numpy as jnp
from jax.experimental import pallas as pl


def kernel(x, edge_index, params):
    raise NotImplementedError("write your pallas kernel here")


# speedup vs baseline: 1.3544x; 1.3544x over previous
import functools

import jax
import jax.numpy as jnp
import numpy as np
from jax import lax
from jax.experimental import pallas as pl
from jax.experimental.pallas import tpu as pltpu
from jax.experimental.pallas import tpu_sc as plsc

N = 1024
E = 65536
D_IN = 128
HID = 32
HEADS = 8
HC = HID * HEADS
HORIZON = 5
W_OUT = HC + 16

BLK = 256
GRID = N // BLK


def _att_mats(gp):
    att_src, att_dst = gp['att_src'], gp['att_dst']
    eye = jnp.eye(HEADS, 16, dtype=jnp.float32)
    A_s = (att_src[:, :, None] * eye[:, None, :]).reshape(HC, 16)
    A_d = (att_dst[:, :, None] * eye[:, None, :]).reshape(HC, 16)
    return A_s, A_d


def _expand_mat():
    R = np.zeros((16, HC), np.float32)
    for h in range(HEADS):
        R[h, h * HID:(h + 1) * HID] = 1.0
    return jnp.asarray(R)


def _k1_body(x_ref, wenc_ref, benc_ref, lng_ref, lnb_ref, w1_ref, as_ref,
             ad_ref, hh_ref, asrc_ref, adst_ref):
    h = jnp.dot(x_ref[...], wenc_ref[...], preferred_element_type=jnp.float32, precision=jax.lax.Precision.HIGHEST)
    h = jnp.maximum(h + benc_ref[...], 0.0)
    mu = jnp.mean(h, axis=-1, keepdims=True)
    var = jnp.mean((h - mu) ** 2, axis=-1, keepdims=True)
    h = (h - mu) * jax.lax.rsqrt(var + 1e-5) * lng_ref[...] + lnb_ref[...]
    hh = jnp.dot(h, w1_ref[...], preferred_element_type=jnp.float32, precision=jax.lax.Precision.HIGHEST)
    hh_ref[...] = hh
    asrc_ref[...] = jnp.dot(hh, as_ref[...], preferred_element_type=jnp.float32, precision=jax.lax.Precision.HIGHEST)
    adst_ref[...] = jnp.dot(hh, ad_ref[...], preferred_element_type=jnp.float32, precision=jax.lax.Precision.HIGHEST)


def _encoder_prep(x, p, A_s, A_d):
    full = lambda shp: pl.BlockSpec(shp, lambda i: (0, 0))
    return pl.pallas_call(
        _k1_body,
        grid=(GRID,),
        in_specs=[
            pl.BlockSpec((BLK, D_IN), lambda i: (i, 0)),
            full((D_IN, HID)), full((1, HID)), full((1, HID)), full((1, HID)),
            full((HID, HC)), full((HC, 16)), full((HC, 16)),
        ],
        out_specs=[
            pl.BlockSpec((BLK, HC), lambda i: (i, 0)),
            pl.BlockSpec((BLK, 16), lambda i: (i, 0)),
            pl.BlockSpec((BLK, 16), lambda i: (i, 0)),
        ],
        out_shape=[
            jax.ShapeDtypeStruct((N, HC), jnp.float32),
            jax.ShapeDtypeStruct((N, 16), jnp.float32),
            jax.ShapeDtypeStruct((N, 16), jnp.float32),
        ],
    )(x, p['enc_W'], p['enc_b'].reshape(1, HID), p['ln_g'].reshape(1, HID),
      p['ln_b'].reshape(1, HID), p['gat'][0]['W'], A_s, A_d)


def _combine_prep_body(outp_ref, r_ref, bias_ref, gprev_ref, w_ref, as_ref,
                       ad_ref, hh_ref, asrc_ref, adst_ref, g_ref, *,
                       residual):
    s = outp_ref[0] + outp_ref[1]
    num = s[:, :HC]
    den16 = s[:, HC:]
    den_e = jnp.dot(den16, r_ref[...], preferred_element_type=jnp.float32, precision=jax.lax.Precision.HIGHEST)
    g = jnp.maximum(num / (den_e + 1e-16) + bias_ref[...], 0.0)
    if residual:
        g = g + gprev_ref[...]
    g_ref[...] = g
    hh = jnp.dot(g, w_ref[...], preferred_element_type=jnp.float32, precision=jax.lax.Precision.HIGHEST)
    hh_ref[...] = hh
    asrc_ref[...] = jnp.dot(hh, as_ref[...], preferred_element_type=jnp.float32, precision=jax.lax.Precision.HIGHEST)
    adst_ref[...] = jnp.dot(hh, ad_ref[...], preferred_element_type=jnp.float32, precision=jax.lax.Precision.HIGHEST)


def _combine_prep(outp, Rm, bias, gprev, W, A_s, A_d, residual):
    full = lambda shp: pl.BlockSpec(shp, lambda i: tuple(0 for _ in shp))
    return pl.pallas_call(
        functools.partial(_combine_prep_body, residual=residual),
        grid=(GRID,),
        in_specs=[
            pl.BlockSpec((2, BLK, W_OUT), lambda i: (0, i, 0)),
            full((16, HC)), full((1, HC)),
            pl.BlockSpec((BLK, HC), lambda i: (i, 0)),
            full((HC, HC)), full((HC, 16)), full((HC, 16)),
        ],
        out_specs=[
            pl.BlockSpec((BLK, HC), lambda i: (i, 0)),
            pl.BlockSpec((BLK, 16), lambda i: (i, 0)),
            pl.BlockSpec((BLK, 16), lambda i: (i, 0)),
            pl.BlockSpec((BLK, HC), lambda i: (i, 0)),
        ],
        out_shape=[
            jax.ShapeDtypeStruct((N, HC), jnp.float32),
            jax.ShapeDtypeStruct((N, 16), jnp.float32),
            jax.ShapeDtypeStruct((N, 16), jnp.float32),
            jax.ShapeDtypeStruct((N, HC), jnp.float32),
        ],
    )(outp, Rm, bias.reshape(1, HC), gprev, W, A_s, A_d)


def _final_body(outp_ref, r_ref, bias_ref, gprev_ref, pw1_ref, pb1_ref,
                pw2_ref, pb2_ref, vw1_ref, vb1_ref, vw2_ref, vb2_ref,
                cwa_ref, cwb_ref, cb1_ref, price_ref, vol_ref, ca_ref,
                cbb_ref, h_ref):
    s = outp_ref[0] + outp_ref[1]
    num = s[:, :HC]
    den16 = s[:, HC:]
    den_e = jnp.dot(den16, r_ref[...], preferred_element_type=jnp.float32, precision=jax.lax.Precision.HIGHEST)
    h = jnp.maximum(num / (den_e + 1e-16) + bias_ref[...], 0.0) + gprev_ref[...]
    h_ref[...] = h
    t = jnp.maximum(jnp.dot(h, pw1_ref[...], preferred_element_type=jnp.float32, precision=jax.lax.Precision.HIGHEST)
                    + pb1_ref[...], 0.0)
    price_ref[...] = jnp.dot(t, pw2_ref[...],
                             preferred_element_type=jnp.float32, precision=jax.lax.Precision.HIGHEST) + pb2_ref[...]
    t = jnp.maximum(jnp.dot(h, vw1_ref[...], preferred_element_type=jnp.float32, precision=jax.lax.Precision.HIGHEST)
                    + vb1_ref[...], 0.0)
    vol_ref[...] = jnp.dot(t, vw2_ref[...],
                           preferred_element_type=jnp.float32, precision=jax.lax.Precision.HIGHEST) + vb2_ref[...]
    ca_ref[...] = jnp.dot(h, cwa_ref[...], preferred_element_type=jnp.float32, precision=jax.lax.Precision.HIGHEST)
    cbb_ref[...] = jnp.dot(h, cwb_ref[...],
                           preferred_element_type=jnp.float32, precision=jax.lax.Precision.HIGHEST) + cb1_ref[...]


def _final_heads(outp, Rm, bias, gprev, p):
    full = lambda shp: pl.BlockSpec(shp, lambda i: tuple(0 for _ in shp))
    return pl.pallas_call(
        _final_body,
        grid=(GRID,),
        in_specs=[
            pl.BlockSpec((2, BLK, W_OUT), lambda i: (0, i, 0)),
            full((16, HC)), full((1, HC)),
            pl.BlockSpec((BLK, HC), lambda i: (i, 0)),
            full((HC, HID)), full((1, HID)), full((HID, HORIZON)),
            full((1, HORIZON)),
            full((HC, HID)), full((1, HID)), full((HID, HORIZON)),
            full((1, HORIZON)),
            full((HC, HID)), full((HC, HID)), full((1, HID)),
        ],
        out_specs=[
            pl.BlockSpec((BLK, HORIZON), lambda i: (i, 0)),
            pl.BlockSpec((BLK, HORIZON), lambda i: (i, 0)),
            pl.BlockSpec((BLK, HID), lambda i: (i, 0)),
            pl.BlockSpec((BLK, HID), lambda i: (i, 0)),
            pl.BlockSpec((BLK, HC), lambda i: (i, 0)),
        ],
        out_shape=[
            jax.ShapeDtypeStruct((N, HORIZON), jnp.float32),
            jax.ShapeDtypeStruct((N, HORIZON), jnp.float32),
            jax.ShapeDtypeStruct((N, HID), jnp.float32),
            jax.ShapeDtypeStruct((N, HID), jnp.float32),
            jax.ShapeDtypeStruct((N, HC), jnp.float32),
        ],
    )(outp, Rm, bias.reshape(1, HC), gprev,
      p['p_W1'], p['p_b1'].reshape(1, HID), p['p_W2'],
      p['p_b2'].reshape(1, HORIZON),
      p['v_W1'], p['v_b1'].reshape(1, HID), p['v_W2'],
      p['v_b2'].reshape(1, HORIZON),
      p['c_W1'][:HC], p['c_W1'][HC:], p['c_b1'].reshape(1, HID))


CORR_BI = 128


def _corr_body(ca_ref, cbt_ref, w2_ref, b2_ref, out_ref):
    acc = jnp.zeros((CORR_BI, N), jnp.float32)
    for k in range(HID):
        t = jnp.maximum(ca_ref[:, k:k + 1] + cbt_ref[k:k + 1, :], 0.0)
        acc = acc + t * w2_ref[k]
    out_ref[...] = jnp.tanh(acc + b2_ref[0])


def _corr(ca, cbT, w2, b2):
    return pl.pallas_call(
        _corr_body,
        grid=(N // CORR_BI,),
        in_specs=[
            pl.BlockSpec((CORR_BI, HID), lambda i: (i, 0)),
            pl.BlockSpec((HID, N), lambda i: (0, 0)),
            pl.BlockSpec(memory_space=pltpu.SMEM),
            pl.BlockSpec(memory_space=pltpu.SMEM),
        ],
        out_specs=pl.BlockSpec((CORR_BI, N), lambda i: (i, 0)),
        out_shape=jax.ShapeDtypeStruct((N, N), jnp.float32),
    )(ca, cbT, w2, b2)


def _edge_phase(src, dst, hh, asrc16, adst16):
    av = asrc16[src] + adst16[dst]
    av = jnp.maximum(av, 0.2 * av)
    pcoef = jnp.exp(av)
    msg = hh[src].reshape(-1, HEADS, HID) * pcoef[:, :HEADS, None]
    num = jax.ops.segment_sum(msg, dst, num_segments=N).reshape(N, HC)
    den = jax.ops.segment_sum(pcoef, dst, num_segments=N)
    out = jnp.concatenate([num, den], axis=1)
    return jnp.stack([out, jnp.zeros_like(out)])


def kernel(x, edge_index, params):
    p = params
    loop = jnp.arange(N, dtype=edge_index.dtype)
    src = jnp.concatenate([edge_index[0], loop])
    dst = jnp.concatenate([edge_index[1], loop])
    Rm = _expand_mat()

    A_s, A_d = _att_mats(p['gat'][0])
    hh, asrc16, adst16 = _encoder_prep(x, p, A_s, A_d)
    outp = _edge_phase(src, dst, hh, asrc16, adst16)

    A_s, A_d = _att_mats(p['gat'][1])
    hh, asrc16, adst16, g1 = _combine_prep(
        outp, Rm, p['gat'][0]['bias'], jnp.zeros((N, HC), jnp.float32),
        p['gat'][1]['W'], A_s, A_d, residual=False)
    outp = _edge_phase(src, dst, hh, asrc16, adst16)

    A_s, A_d = _att_mats(p['gat'][2])
    hh, asrc16, adst16, g2 = _combine_prep(
        outp, Rm, p['gat'][1]['bias'], g1, p['gat'][2]['W'], A_s, A_d,
        residual=True)
    outp = _edge_phase(src, dst, hh, asrc16, adst16)

    price, vol, ca, cbb, h = _final_heads(outp, Rm, p['gat'][2]['bias'], g2, p)

    corr = _corr(ca, cbb.T, p['c_W2'][:, 0], p['c_b2'])
    return (price, vol, corr, h)


# speedup vs baseline: 13.6125x; 10.0502x over previous
import functools

import jax
import jax.numpy as jnp
import numpy as np
from jax import lax
from jax.experimental import pallas as pl
from jax.experimental.pallas import tpu as pltpu
from jax.experimental.pallas import tpu_sc as plsc

N = 1024
E = 65536
D_IN = 128
HID = 32
HEADS = 8
HC = HID * HEADS
HORIZON = 5

BLK = 256
GRID = N // BLK

HIGHEST = jax.lax.Precision.HIGHEST

ETOT = E + N
NC, NS = 2, 16
NSH, NG = 4, 4
GW = HC // NG
ESH = ETOT // (NC * NSH)
EC = 80
NCHUNK = ESH // EC


def _prep_weights(gp):
    W = gp['W']
    d = W.shape[0]
    att_src, att_dst = gp['att_src'], gp['att_dst']
    eye = jnp.eye(HEADS, 16, dtype=jnp.float32)
    A_s = (att_src[:, :, None] * eye[:, None, :]).reshape(HC, 16)
    A_d = (att_dst[:, :, None] * eye[:, None, :]).reshape(HC, 16)
    z = jnp.zeros((d, 128 - GW), jnp.float32)
    W4 = jnp.concatenate(
        [jnp.concatenate([W[:, g * GW:(g + 1) * GW], z], axis=1)
         for g in range(NG)], axis=1)
    zr = jnp.zeros((NG, 128 - GW, 16), jnp.float32)
    A_s4 = jnp.concatenate([A_s.reshape(NG, GW, 16), zr], axis=1).reshape(512, 16)
    A_d4 = jnp.concatenate([A_d.reshape(NG, GW, 16), zr], axis=1).reshape(512, 16)
    return W4, A_s4, A_d4


def _expand_mat():
    R = np.zeros((16, HC), np.float32)
    for h in range(HEADS):
        R[h, h * HID:(h + 1) * HID] = 0.25
    return jnp.asarray(R)


def _k1_body(x_ref, wenc_ref, benc_ref, lng_ref, lnb_ref, w4_ref, was_ref,
             wad_ref, hh4_ref, asrc_ref, adst_ref):
    h = jnp.dot(x_ref[...].astype(jnp.bfloat16),
                wenc_ref[...].astype(jnp.bfloat16),
                preferred_element_type=jnp.float32)
    h = jnp.maximum(h + benc_ref[...], 0.0)
    mu = jnp.mean(h, axis=-1, keepdims=True)
    var = jnp.mean((h - mu) ** 2, axis=-1, keepdims=True)
    h = (h - mu) * jax.lax.rsqrt(var + 1e-5) * lng_ref[...] + lnb_ref[...]
    hh4 = jnp.dot(h.astype(jnp.bfloat16), w4_ref[...].astype(jnp.bfloat16),
                  preferred_element_type=jnp.float32)
    hh4_ref[...] = hh4
    asrc_ref[...] = jnp.dot(hh4, was_ref[...],
                            preferred_element_type=jnp.float32,
                            precision=HIGHEST)
    adst_ref[...] = jnp.dot(hh4, wad_ref[...],
                            preferred_element_type=jnp.float32,
                            precision=HIGHEST)


def _encoder_prep(x, p, W4, WAs, WAd):
    full = lambda shp: pl.BlockSpec(shp, lambda i: (0, 0))
    return pl.pallas_call(
        _k1_body,
        grid=(GRID,),
        in_specs=[
            pl.BlockSpec((BLK, D_IN), lambda i: (i, 0)),
            full((D_IN, HID)), full((1, HID)), full((1, HID)), full((1, HID)),
            full((HID, 512)), full((512, 16)), full((512, 16)),
        ],
        out_specs=[
            pl.BlockSpec((BLK, 512), lambda i: (i, 0)),
            pl.BlockSpec((BLK, 16), lambda i: (i, 0)),
            pl.BlockSpec((BLK, 16), lambda i: (i, 0)),
        ],
        out_shape=[
            jax.ShapeDtypeStruct((N, 512), jnp.float32),
            jax.ShapeDtypeStruct((N, 16), jnp.float32),
            jax.ShapeDtypeStruct((N, 16), jnp.float32),
        ],
    )(x, p['enc_W'], p['enc_b'].reshape(1, HID), p['ln_g'].reshape(1, HID),
      p['ln_b'].reshape(1, HID), W4, WAs, WAd)


def _combined_h(msum_ref, dsum_ref, r_ref, bias_ref):
    m = jnp.sum(msum_ref[...], axis=0)
    num = jnp.concatenate(
        [m[g] + m[NG + g] + m[2 * NG + g] + m[3 * NG + g] for g in range(NG)],
        axis=-1)
    den16 = jnp.sum(dsum_ref[...], axis=(0, 1))
    den_e = jnp.dot(den16, r_ref[...], preferred_element_type=jnp.float32,
                    precision=HIGHEST)
    return jnp.maximum(num / (den_e + 1e-16) + bias_ref[...], 0.0)


def _combine_prep_body(msum_ref, dsum_ref, r_ref, bias_ref, gprev_ref,
                       w4_ref, was_ref, wad_ref, hh4_ref, asrc_ref, adst_ref,
                       g_ref, *, residual):
    g = _combined_h(msum_ref, dsum_ref, r_ref, bias_ref)
    if residual:
        g = g + gprev_ref[...]
    g_ref[...] = g
    hh4 = jnp.dot(g.astype(jnp.bfloat16), w4_ref[...].astype(jnp.bfloat16),
                  preferred_element_type=jnp.float32)
    hh4_ref[...] = hh4
    asrc_ref[...] = jnp.dot(hh4, was_ref[...],
                            preferred_element_type=jnp.float32,
                            precision=HIGHEST)
    adst_ref[...] = jnp.dot(hh4, wad_ref[...],
                            preferred_element_type=jnp.float32,
                            precision=HIGHEST)


def _combine_prep(msum, dsum, Rm, bias, gprev, W4, WAs, WAd, residual):
    full = lambda shp: pl.BlockSpec(shp, lambda i: tuple(0 for _ in shp))
    return pl.pallas_call(
        functools.partial(_combine_prep_body, residual=residual),
        grid=(GRID,),
        in_specs=[
            pl.BlockSpec((NC, NS, BLK, GW), lambda i: (0, 0, i, 0)),
            pl.BlockSpec((NC, NS, BLK, 16), lambda i: (0, 0, i, 0)),
            full((16, HC)), full((1, HC)),
            pl.BlockSpec((BLK, HC), lambda i: (i, 0)),
            full((HC, 512)), full((512, 16)), full((512, 16)),
        ],
        out_specs=[
            pl.BlockSpec((BLK, 512), lambda i: (i, 0)),
            pl.BlockSpec((BLK, 16), lambda i: (i, 0)),
            pl.BlockSpec((BLK, 16), lambda i: (i, 0)),
            pl.BlockSpec((BLK, HC), lambda i: (i, 0)),
        ],
        out_shape=[
            jax.ShapeDtypeStruct((N, 512), jnp.float32),
            jax.ShapeDtypeStruct((N, 16), jnp.float32),
            jax.ShapeDtypeStruct((N, 16), jnp.float32),
            jax.ShapeDtypeStruct((N, HC), jnp.float32),
        ],
    )(msum, dsum, Rm, bias.reshape(1, HC), gprev, W4, WAs, WAd)


def _final_body(msum_ref, dsum_ref, r_ref, bias_ref, gprev_ref, pw1_ref,
                pb1_ref, pw2_ref, pb2_ref, vw1_ref, vb1_ref, vw2_ref,
                vb2_ref, cwa_ref, cwb_ref, cb1_ref, price_ref, vol_ref,
                ca_ref, cbb_ref, h_ref):
    h = _combined_h(msum_ref, dsum_ref, r_ref, bias_ref) + gprev_ref[...]
    h_ref[...] = h
    hb = h.astype(jnp.bfloat16)
    t = jnp.maximum(jnp.dot(hb, pw1_ref[...].astype(jnp.bfloat16),
                            preferred_element_type=jnp.float32)
                    + pb1_ref[...], 0.0)
    price_ref[...] = jnp.dot(t.astype(jnp.bfloat16),
                             pw2_ref[...].astype(jnp.bfloat16),
                             preferred_element_type=jnp.float32) + pb2_ref[...]
    t = jnp.maximum(jnp.dot(hb, vw1_ref[...].astype(jnp.bfloat16),
                            preferred_element_type=jnp.float32)
                    + vb1_ref[...], 0.0)
    vol_ref[...] = jnp.dot(t.astype(jnp.bfloat16),
                           vw2_ref[...].astype(jnp.bfloat16),
                           preferred_element_type=jnp.float32) + vb2_ref[...]
    ca_ref[...] = jnp.dot(hb, cwa_ref[...].astype(jnp.bfloat16),
                          preferred_element_type=jnp.float32)
    cbb_ref[...] = jnp.dot(hb, cwb_ref[...].astype(jnp.bfloat16),
                           preferred_element_type=jnp.float32) + cb1_ref[...]


def _final_heads(msum, dsum, Rm, bias, gprev, p):
    full = lambda shp: pl.BlockSpec(shp, lambda i: tuple(0 for _ in shp))
    return pl.pallas_call(
        _final_body,
        grid=(GRID,),
        in_specs=[
            pl.BlockSpec((NC, NS, BLK, GW), lambda i: (0, 0, i, 0)),
            pl.BlockSpec((NC, NS, BLK, 16), lambda i: (0, 0, i, 0)),
            full((16, HC)), full((1, HC)),
            pl.BlockSpec((BLK, HC), lambda i: (i, 0)),
            full((HC, HID)), full((1, HID)), full((HID, HORIZON)),
            full((1, HORIZON)),
            full((HC, HID)), full((1, HID)), full((HID, HORIZON)),
            full((1, HORIZON)),
            full((HC, HID)), full((HC, HID)), full((1, HID)),
        ],
        out_specs=[
            pl.BlockSpec((BLK, HORIZON), lambda i: (i, 0)),
            pl.BlockSpec((BLK, HORIZON), lambda i: (i, 0)),
            pl.BlockSpec((BLK, HID), lambda i: (i, 0)),
            pl.BlockSpec((BLK, HID), lambda i: (i, 0)),
            pl.BlockSpec((BLK, HC), lambda i: (i, 0)),
        ],
        out_shape=[
            jax.ShapeDtypeStruct((N, HORIZON), jnp.float32),
            jax.ShapeDtypeStruct((N, HORIZON), jnp.float32),
            jax.ShapeDtypeStruct((N, HID), jnp.float32),
            jax.ShapeDtypeStruct((N, HID), jnp.float32),
            jax.ShapeDtypeStruct((N, HC), jnp.float32),
        ],
    )(msum, dsum, Rm, bias.reshape(1, HC), gprev,
      p['p_W1'], p['p_b1'].reshape(1, HID), p['p_W2'],
      p['p_b2'].reshape(1, HORIZON),
      p['v_W1'], p['v_b1'].reshape(1, HID), p['v_W2'],
      p['v_b2'].reshape(1, HORIZON),
      p['c_W1'][:HC], p['c_W1'][HC:], p['c_b1'].reshape(1, HID))


CORR_BI = 128


def _corr_body(ca_ref, cbt_ref, w2_ref, b2_ref, out_ref):
    acc = jnp.zeros((CORR_BI, N), jnp.float32)
    for k in range(HID):
        t = jnp.maximum(ca_ref[:, k:k + 1] + cbt_ref[k:k + 1, :], 0.0)
        t = t.astype(jnp.bfloat16).astype(jnp.float32)
        acc = acc + t * w2_ref[k]
    out_ref[...] = jnp.tanh(acc + b2_ref[0])


def _corr(ca, cbT, w2, b2):
    return pl.pallas_call(
        _corr_body,
        grid=(N // CORR_BI,),
        in_specs=[
            pl.BlockSpec((CORR_BI, HID), lambda i: (i, 0)),
            pl.BlockSpec((HID, N), lambda i: (0, 0)),
            pl.BlockSpec(memory_space=pltpu.SMEM),
            pl.BlockSpec(memory_space=pltpu.SMEM),
        ],
        out_specs=pl.BlockSpec((CORR_BI, N), lambda i: (i, 0)),
        out_shape=jax.ShapeDtypeStruct((N, N), jnp.float32),
    )(ca, cbT, w2, b2)


def _edge_body(src_hbm, dst_hbm, hh4_hbm, asrc_hbm, adst_hbm, zeros_hbm,
               msum_hbm, dsum_hbm, srcv, dstv, hhbuf, ast, adt, acc, dacc,
               sem):
    c = lax.axis_index("c")
    s = lax.axis_index("s")
    es = s // NG
    g = s % NG

    pltpu.sync_copy(asrc_hbm, ast)
    pltpu.sync_copy(adst_hbm, adt)
    pltpu.sync_copy(zeros_hbm, acc)
    pltpu.sync_copy(zeros_hbm.at[pl.ds(0, N * 16)], dacc)

    idx_lo = jnp.full((16,), 2 * g, jnp.int32)
    idx_hi = jnp.full((16,), 2 * g + 1, jnp.int32)

    def chunk(t, carry):
        base = (c * NSH + es) * ESH + t * EC
        pltpu.sync_copy(src_hbm.at[pl.ds(base, EC)], srcv)
        pltpu.sync_copy(dst_hbm.at[pl.ds(base, EC)], dstv)
        pltpu.async_copy(hh4_hbm.at[g].at[srcv], hhbuf, sem).wait()

        def block16(jb, carry2):
            j0 = jb * 16
            sv16 = srcv[pl.ds(j0, 16)] * 16
            dv16 = dstv[pl.ds(j0, 16)]
            dv16t = dv16 * 16
            dv64 = dv16 * GW
            for r in range(16):
                j = j0 + r
                av = ast[pl.ds(sv16[r], 16)] + adt[pl.ds(dv16t[r], 16)]
                av = jnp.maximum(av, 0.2 * av)
                pv = jnp.exp(av)
                plsc.addupdate(dacc.at[pl.ds(dv16t[r], 16)], pv)
                blo = pv.at[idx_lo].get(mode='promise_in_bounds')
                bhi = pv.at[idx_hi].get(mode='promise_in_bounds')
                rb = dv64[r]
                plsc.addupdate(acc.at[pl.ds(rb, 16)],
                               hhbuf[j, pl.ds(0, 16)] * blo)
                plsc.addupdate(acc.at[pl.ds(rb + 16, 16)],
                               hhbuf[j, pl.ds(16, 16)] * blo)
                plsc.addupdate(acc.at[pl.ds(rb + 32, 16)],
                               hhbuf[j, pl.ds(32, 16)] * bhi)
                plsc.addupdate(acc.at[pl.ds(rb + 48, 16)],
                               hhbuf[j, pl.ds(48, 16)] * bhi)
            return carry2

        lax.fori_loop(0, EC // 16, block16, 0)
        return carry

    lax.fori_loop(0, NCHUNK, chunk, 0)
    pltpu.sync_copy(acc, msum_hbm.at[c, s])
    pltpu.sync_copy(dacc, dsum_hbm.at[c, s])


@jax.jit
def _edge_phase_sc(src, dst, hh4, asrc_flat, adst_flat, zeros_flat):
    mesh = plsc.VectorSubcoreMesh(core_axis_name="c", subcore_axis_name="s")
    fn = pl.kernel(
        _edge_body,
        out_type=[
            jax.ShapeDtypeStruct((NC, NS, N * GW), jnp.float32),
            jax.ShapeDtypeStruct((NC, NS, N * 16), jnp.float32),
        ],
        mesh=mesh,
        scratch_types=[
            pltpu.VMEM((EC,), jnp.int32),
            pltpu.VMEM((EC,), jnp.int32),
            pltpu.VMEM((EC, 128), jnp.float32),
            pltpu.VMEM((N * 16,), jnp.float32),
            pltpu.VMEM((N * 16,), jnp.float32),
            pltpu.VMEM((N * GW,), jnp.float32),
            pltpu.VMEM((N * 16,), jnp.float32),
            pltpu.SemaphoreType.DMA,
        ],
    )
    return fn(src, dst, hh4, asrc_flat, adst_flat, zeros_flat)


def _edge_phase(src, dst, hh512, asrc16, adst16):
    hh4 = hh512.reshape(N, NG, 128).transpose(1, 0, 2)
    zeros_flat = jnp.zeros((N * GW,), jnp.float32)
    msum, dsum = _edge_phase_sc(src, dst, hh4, asrc16.reshape(-1),
                                adst16.reshape(-1), zeros_flat)
    return msum.reshape(NC, NS, N, GW), dsum.reshape(NC, NS, N, 16)


def kernel(x, edge_index, params):
    p = params
    loop = jnp.arange(N, dtype=edge_index.dtype)
    src = jnp.concatenate([edge_index[0], loop])
    dst = jnp.concatenate([edge_index[1], loop])
    Rm = _expand_mat()

    W4, WAs, WAd = _prep_weights(p['gat'][0])
    hh512, asrc16, adst16 = _encoder_prep(x, p, W4, WAs, WAd)
    msum, dsum = _edge_phase(src, dst, hh512, asrc16, adst16)

    W4, WAs, WAd = _prep_weights(p['gat'][1])
    hh512, asrc16, adst16, g1 = _combine_prep(
        msum, dsum, Rm, p['gat'][0]['bias'], jnp.zeros((N, HC), jnp.float32),
        W4, WAs, WAd, residual=False)
    msum, dsum = _edge_phase(src, dst, hh512, asrc16, adst16)

    W4, WAs, WAd = _prep_weights(p['gat'][2])
    hh512, asrc16, adst16, g2 = _combine_prep(
        msum, dsum, Rm, p['gat'][1]['bias'], g1, W4, WAs, WAd, residual=True)
    msum, dsum = _edge_phase(src, dst, hh512, asrc16, adst16)

    price, vol, ca, cbb, h = _final_heads(msum, dsum, Rm, p['gat'][2]['bias'],
                                          g2, p)
    w2r = p['c_W2'][:, 0].astype(jnp.bfloat16).astype(jnp.float32)
    corr = _corr(ca, cbb.T, w2r, p['c_b2'])
    return (price, vol, corr, h)


# speedup vs baseline: 21.1134x; 1.5510x over previous
import functools

import jax
import jax.numpy as jnp
import numpy as np
from jax import lax
from jax.experimental import pallas as pl
from jax.experimental.pallas import tpu as pltpu
from jax.experimental.pallas import tpu_sc as plsc

N = 1024
E = 65536
D_IN = 128
HID = 32
HEADS = 8
HC = HID * HEADS
HORIZON = 5

BLK = 256
GRID = N // BLK

HIGHEST = jax.lax.Precision.HIGHEST

ETOT = E + N
NC, NS = 2, 16
NSH, NG = 4, 4
GW = HC // NG
ESH = ETOT // (NC * NSH)
EC = 80
NCHUNK = ESH // EC


def _prep_weights(gp):
    W = gp['W']
    d = W.shape[0]
    att_src, att_dst = gp['att_src'], gp['att_dst']
    eye = jnp.eye(HEADS, 16, dtype=jnp.float32)
    A_s = (att_src[:, :, None] * eye[:, None, :]).reshape(HC, 16)
    A_d = (att_dst[:, :, None] * eye[:, None, :]).reshape(HC, 16)
    z = jnp.zeros((d, 128 - GW), jnp.float32)
    W4 = jnp.concatenate(
        [jnp.concatenate([W[:, g * GW:(g + 1) * GW], z], axis=1)
         for g in range(NG)], axis=1)
    zr = jnp.zeros((NG, 128 - GW, 16), jnp.float32)
    A_s4 = jnp.concatenate([A_s.reshape(NG, GW, 16), zr], axis=1).reshape(512, 16)
    A_d4 = jnp.concatenate([A_d.reshape(NG, GW, 16), zr], axis=1).reshape(512, 16)
    P = np.zeros((16, 512), np.float32)
    for g in range(NG):
        for k in range(16):
            P[k, 128 * g + GW + k] = 1.0
    return W4, A_s4, A_d4, jnp.asarray(P)


def _expand_mat():
    R = np.zeros((16, HC), np.float32)
    for h in range(HEADS):
        R[h, h * HID:(h + 1) * HID] = 0.25
    return jnp.asarray(R)


def _k1_body(x_ref, wenc_ref, benc_ref, lng_ref, lnb_ref, w4_ref, was_ref,
             wad_ref, p_ref, hh4_ref, adst_ref):
    h = jnp.dot(x_ref[...].astype(jnp.bfloat16),
                wenc_ref[...].astype(jnp.bfloat16),
                preferred_element_type=jnp.float32)
    h = jnp.maximum(h + benc_ref[...], 0.0)
    mu = jnp.mean(h, axis=-1, keepdims=True)
    var = jnp.mean((h - mu) ** 2, axis=-1, keepdims=True)
    h = (h - mu) * jax.lax.rsqrt(var + 1e-5) * lng_ref[...] + lnb_ref[...]
    hh4 = jnp.dot(h.astype(jnp.bfloat16), w4_ref[...].astype(jnp.bfloat16),
                  preferred_element_type=jnp.float32)
    asrc = jnp.dot(hh4, was_ref[...], preferred_element_type=jnp.float32,
                   precision=HIGHEST)
    hh4_ref[...] = hh4 + jnp.dot(asrc, p_ref[...],
                                 preferred_element_type=jnp.float32,
                                 precision=HIGHEST)
    adst_ref[...] = jnp.dot(hh4, wad_ref[...],
                            preferred_element_type=jnp.float32,
                            precision=HIGHEST)


def _encoder_prep(x, p, W4, WAs, WAd, Pm):
    full = lambda shp: pl.BlockSpec(shp, lambda i: (0, 0))
    return pl.pallas_call(
        _k1_body,
        grid=(GRID,),
        in_specs=[
            pl.BlockSpec((BLK, D_IN), lambda i: (i, 0)),
            full((D_IN, HID)), full((1, HID)), full((1, HID)), full((1, HID)),
            full((HID, 512)), full((512, 16)), full((512, 16)),
            full((16, 512)),
        ],
        out_specs=[
            pl.BlockSpec((BLK, 512), lambda i: (i, 0)),
            pl.BlockSpec((BLK, 16), lambda i: (i, 0)),
        ],
        out_shape=[
            jax.ShapeDtypeStruct((N, 512), jnp.float32),
            jax.ShapeDtypeStruct((N, 16), jnp.float32),
        ],
    )(x, p['enc_W'], p['enc_b'].reshape(1, HID), p['ln_g'].reshape(1, HID),
      p['ln_b'].reshape(1, HID), W4, WAs, WAd, Pm)


def _combined_h(msum_ref, dsum_ref, r_ref, bias_ref):
    m = jnp.sum(msum_ref[...], axis=0)
    num = jnp.concatenate(
        [m[g] + m[NG + g] + m[2 * NG + g] + m[3 * NG + g] for g in range(NG)],
        axis=-1)
    den16 = jnp.sum(dsum_ref[...], axis=(0, 1))
    den_e = jnp.dot(den16, r_ref[...], preferred_element_type=jnp.float32,
                    precision=HIGHEST)
    return jnp.maximum(num / (den_e + 1e-16) + bias_ref[...], 0.0)


def _combine_prep_body(msum_ref, dsum_ref, r_ref, bias_ref, gprev_ref,
                       w4_ref, was_ref, wad_ref, p_ref, hh4_ref, adst_ref,
                       g_ref, *, residual):
    g = _combined_h(msum_ref, dsum_ref, r_ref, bias_ref)
    if residual:
        g = g + gprev_ref[...]
    g_ref[...] = g
    hh4 = jnp.dot(g.astype(jnp.bfloat16), w4_ref[...].astype(jnp.bfloat16),
                  preferred_element_type=jnp.float32)
    asrc = jnp.dot(hh4, was_ref[...], preferred_element_type=jnp.float32,
                   precision=HIGHEST)
    hh4_ref[...] = hh4 + jnp.dot(asrc, p_ref[...],
                                 preferred_element_type=jnp.float32,
                                 precision=HIGHEST)
    adst_ref[...] = jnp.dot(hh4, wad_ref[...],
                            preferred_element_type=jnp.float32,
                            precision=HIGHEST)


def _combine_prep(msum, dsum, Rm, bias, gprev, W4, WAs, WAd, Pm, residual):
    full = lambda shp: pl.BlockSpec(shp, lambda i: tuple(0 for _ in shp))
    return pl.pallas_call(
        functools.partial(_combine_prep_body, residual=residual),
        grid=(GRID,),
        in_specs=[
            pl.BlockSpec((NC, NS, BLK, GW), lambda i: (0, 0, i, 0)),
            pl.BlockSpec((NC, NS, BLK, 16), lambda i: (0, 0, i, 0)),
            full((16, HC)), full((1, HC)),
            pl.BlockSpec((BLK, HC), lambda i: (i, 0)),
            full((HC, 512)), full((512, 16)), full((512, 16)),
            full((16, 512)),
        ],
        out_specs=[
            pl.BlockSpec((BLK, 512), lambda i: (i, 0)),
            pl.BlockSpec((BLK, 16), lambda i: (i, 0)),
            pl.BlockSpec((BLK, HC), lambda i: (i, 0)),
        ],
        out_shape=[
            jax.ShapeDtypeStruct((N, 512), jnp.float32),
            jax.ShapeDtypeStruct((N, 16), jnp.float32),
            jax.ShapeDtypeStruct((N, HC), jnp.float32),
        ],
    )(msum, dsum, Rm, bias.reshape(1, HC), gprev, W4, WAs, WAd, Pm)


def _final_body(msum_ref, dsum_ref, r_ref, bias_ref, gprev_ref, pw1_ref,
                pb1_ref, pw2_ref, pb2_ref, vw1_ref, vb1_ref, vw2_ref,
                vb2_ref, cwa_ref, cwb_ref, cb1_ref, price_ref, vol_ref,
                ca_ref, cbb_ref, h_ref):
    h = _combined_h(msum_ref, dsum_ref, r_ref, bias_ref) + gprev_ref[...]
    h_ref[...] = h
    hb = h.astype(jnp.bfloat16)
    t = jnp.maximum(jnp.dot(hb, pw1_ref[...].astype(jnp.bfloat16),
                            preferred_element_type=jnp.float32)
                    + pb1_ref[...], 0.0)
    price_ref[...] = jnp.dot(t.astype(jnp.bfloat16),
                             pw2_ref[...].astype(jnp.bfloat16),
                             preferred_element_type=jnp.float32) + pb2_ref[...]
    t = jnp.maximum(jnp.dot(hb, vw1_ref[...].astype(jnp.bfloat16),
                            preferred_element_type=jnp.float32)
                    + vb1_ref[...], 0.0)
    vol_ref[...] = jnp.dot(t.astype(jnp.bfloat16),
                           vw2_ref[...].astype(jnp.bfloat16),
                           preferred_element_type=jnp.float32) + vb2_ref[...]
    ca_ref[...] = jnp.dot(hb, cwa_ref[...].astype(jnp.bfloat16),
                          preferred_element_type=jnp.float32)
    cbb_ref[...] = jnp.dot(hb, cwb_ref[...].astype(jnp.bfloat16),
                           preferred_element_type=jnp.float32) + cb1_ref[...]


def _final_heads(msum, dsum, Rm, bias, gprev, p):
    full = lambda shp: pl.BlockSpec(shp, lambda i: tuple(0 for _ in shp))
    return pl.pallas_call(
        _final_body,
        grid=(GRID,),
        in_specs=[
            pl.BlockSpec((NC, NS, BLK, GW), lambda i: (0, 0, i, 0)),
            pl.BlockSpec((NC, NS, BLK, 16), lambda i: (0, 0, i, 0)),
            full((16, HC)), full((1, HC)),
            pl.BlockSpec((BLK, HC), lambda i: (i, 0)),
            full((HC, HID)), full((1, HID)), full((HID, HORIZON)),
            full((1, HORIZON)),
            full((HC, HID)), full((1, HID)), full((HID, HORIZON)),
            full((1, HORIZON)),
            full((HC, HID)), full((HC, HID)), full((1, HID)),
        ],
        out_specs=[
            pl.BlockSpec((BLK, HORIZON), lambda i: (i, 0)),
            pl.BlockSpec((BLK, HORIZON), lambda i: (i, 0)),
            pl.BlockSpec((BLK, HID), lambda i: (i, 0)),
            pl.BlockSpec((BLK, HID), lambda i: (i, 0)),
            pl.BlockSpec((BLK, HC), lambda i: (i, 0)),
        ],
        out_shape=[
            jax.ShapeDtypeStruct((N, HORIZON), jnp.float32),
            jax.ShapeDtypeStruct((N, HORIZON), jnp.float32),
            jax.ShapeDtypeStruct((N, HID), jnp.float32),
            jax.ShapeDtypeStruct((N, HID), jnp.float32),
            jax.ShapeDtypeStruct((N, HC), jnp.float32),
        ],
    )(msum, dsum, Rm, bias.reshape(1, HC), gprev,
      p['p_W1'], p['p_b1'].reshape(1, HID), p['p_W2'],
      p['p_b2'].reshape(1, HORIZON),
      p['v_W1'], p['v_b1'].reshape(1, HID), p['v_W2'],
      p['v_b2'].reshape(1, HORIZON),
      p['c_W1'][:HC], p['c_W1'][HC:], p['c_b1'].reshape(1, HID))


CORR_BI = 128


def _corr_body(ca_ref, cbt_ref, w2_ref, b2_ref, out_ref):
    acc = jnp.zeros((CORR_BI, N), jnp.float32)
    for k in range(HID):
        t = jnp.maximum(ca_ref[:, k:k + 1] + cbt_ref[k:k + 1, :], 0.0)
        t = t.astype(jnp.bfloat16).astype(jnp.float32)
        acc = acc + t * w2_ref[k]
    out_ref[...] = jnp.tanh(acc + b2_ref[0])


def _corr(ca, cbT, w2, b2):
    return pl.pallas_call(
        _corr_body,
        grid=(N // CORR_BI,),
        in_specs=[
            pl.BlockSpec((CORR_BI, HID), lambda i: (i, 0)),
            pl.BlockSpec((HID, N), lambda i: (0, 0)),
            pl.BlockSpec(memory_space=pltpu.SMEM),
            pl.BlockSpec(memory_space=pltpu.SMEM),
        ],
        out_specs=pl.BlockSpec((CORR_BI, N), lambda i: (i, 0)),
        out_shape=jax.ShapeDtypeStruct((N, N), jnp.float32),
    )(ca, cbT, w2, b2)


def _edge_body(pk_hbm, hh4_hbm, adst_hbm, zeros_hbm, msum_hbm, dsum_hbm,
               pkall, srcv0, srcv1, hhbuf0, hhbuf1, adt, acc, dacc,
               sem0, sem1):
    c = lax.axis_index("c")
    s = lax.axis_index("s")
    es = s // NG
    g = s % NG
    ebase = (c * NSH + es) * ESH

    pltpu.sync_copy(pk_hbm.at[pl.ds(ebase, ESH)], pkall)
    pltpu.sync_copy(adst_hbm, adt)
    pltpu.sync_copy(zeros_hbm, acc)
    pltpu.sync_copy(zeros_hbm.at[pl.ds(0, N * 16)], dacc)

    idx_lo = jnp.full((16,), 2 * g, jnp.int32)
    idx_hi = jnp.full((16,), 2 * g + 1, jnp.int32)

    def fetch(t, srcv, hhbuf, sem):
        for b in range(EC // 16):
            v = pkall[pl.ds(t * EC + b * 16, 16)]
            srcv[pl.ds(b * 16, 16)] = v & 65535
        return pltpu.async_copy(hh4_hbm.at[g].at[srcv], hhbuf, sem)

    def compute(t, hhbuf):
        def block16(jb, carry2):
            j0 = jb * 16
            dv16 = pkall[pl.ds(t * EC + j0, 16)] >> 16
            dv16t = dv16 * 16
            dv64 = dv16 * GW
            for r in range(16):
                j = j0 + r
                av = hhbuf[j, pl.ds(GW, 16)] + adt[pl.ds(dv16t[r], 16)]
                av = jnp.maximum(av, 0.2 * av)
                pv = jnp.exp(av)
                plsc.addupdate(dacc.at[pl.ds(dv16t[r], 16)], pv)
                blo = pv.at[idx_lo].get(mode='promise_in_bounds')
                bhi = pv.at[idx_hi].get(mode='promise_in_bounds')
                rb = dv64[r]
                plsc.addupdate(acc.at[pl.ds(rb, 16)],
                               hhbuf[j, pl.ds(0, 16)] * blo)
                plsc.addupdate(acc.at[pl.ds(rb + 16, 16)],
                               hhbuf[j, pl.ds(16, 16)] * blo)
                plsc.addupdate(acc.at[pl.ds(rb + 32, 16)],
                               hhbuf[j, pl.ds(32, 16)] * bhi)
                plsc.addupdate(acc.at[pl.ds(rb + 48, 16)],
                               hhbuf[j, pl.ds(48, 16)] * bhi)
            return carry2

        lax.fori_loop(0, EC // 16, block16, 0)

    fetch(0, srcv0, hhbuf0, sem0)

    def pair(u, carry):
        a = 2 * u
        fetch(a + 1, srcv1, hhbuf1, sem1)
        pltpu.make_async_copy(hh4_hbm.at[g].at[srcv0], hhbuf0, sem0).wait()
        compute(a, hhbuf0)
        nxt = jnp.where(a + 2 >= NCHUNK, 0, a + 2)
        fetch(nxt, srcv0, hhbuf0, sem0)
        pltpu.make_async_copy(hh4_hbm.at[g].at[srcv1], hhbuf1, sem1).wait()
        compute(a + 1, hhbuf1)
        return carry

    lax.fori_loop(0, NCHUNK // 2, pair, 0)
    pltpu.make_async_copy(hh4_hbm.at[g].at[srcv0], hhbuf0, sem0).wait()
    pltpu.sync_copy(acc, msum_hbm.at[c, s])
    pltpu.sync_copy(dacc, dsum_hbm.at[c, s])


@jax.jit
def _edge_phase_sc(pk, hh4, adst_flat, zeros_flat):
    mesh = plsc.VectorSubcoreMesh(core_axis_name="c", subcore_axis_name="s")
    fn = pl.kernel(
        _edge_body,
        out_type=[
            jax.ShapeDtypeStruct((NC, NS, N * GW), jnp.float32),
            jax.ShapeDtypeStruct((NC, NS, N * 16), jnp.float32),
        ],
        mesh=mesh,
        scratch_types=[
            pltpu.VMEM((ESH,), jnp.int32),
            pltpu.VMEM((EC,), jnp.int32),
            pltpu.VMEM((EC,), jnp.int32),
            pltpu.VMEM((EC, 128), jnp.float32),
            pltpu.VMEM((EC, 128), jnp.float32),
            pltpu.VMEM((N * 16,), jnp.float32),
            pltpu.VMEM((N * GW,), jnp.float32),
            pltpu.VMEM((N * 16,), jnp.float32),
            pltpu.SemaphoreType.DMA,
            pltpu.SemaphoreType.DMA,
        ],
    )
    return fn(pk, hh4, adst_flat, zeros_flat)


def _edge_phase(pk, hh512, adst16):
    hh4 = hh512.reshape(N, NG, 128).transpose(1, 0, 2)
    zeros_flat = jnp.zeros((N * GW,), jnp.float32)
    msum, dsum = _edge_phase_sc(pk, hh4, adst16.reshape(-1), zeros_flat)
    return msum.reshape(NC, NS, N, GW), dsum.reshape(NC, NS, N, 16)


def kernel(x, edge_index, params):
    p = params
    loop = jnp.arange(N, dtype=edge_index.dtype)
    src = jnp.concatenate([edge_index[0], loop])
    dst = jnp.concatenate([edge_index[1], loop])
    pk = src | (dst << 16)
    Rm = _expand_mat()

    W4, WAs, WAd, Pm = _prep_weights(p['gat'][0])
    hh512, adst16 = _encoder_prep(x, p, W4, WAs, WAd, Pm)
    msum, dsum = _edge_phase(pk, hh512, adst16)

    W4, WAs, WAd, Pm = _prep_weights(p['gat'][1])
    hh512, adst16, g1 = _combine_prep(
        msum, dsum, Rm, p['gat'][0]['bias'], jnp.zeros((N, HC), jnp.float32),
        W4, WAs, WAd, Pm, residual=False)
    msum, dsum = _edge_phase(pk, hh512, adst16)

    W4, WAs, WAd, Pm = _prep_weights(p['gat'][2])
    hh512, adst16, g2 = _combine_prep(
        msum, dsum, Rm, p['gat'][1]['bias'], g1, W4, WAs, WAd, Pm,
        residual=True)
    msum, dsum = _edge_phase(pk, hh512, adst16)

    price, vol, ca, cbb, h = _final_heads(msum, dsum, Rm, p['gat'][2]['bias'],
                                          g2, p)
    w2r = p['c_W2'][:, 0].astype(jnp.bfloat16).astype(jnp.float32)
    corr = _corr(ca, cbb.T, w2r, p['c_b2'])
    return (price, vol, corr, h)
